# Initial kernel scaffold; baseline (speedup 1.0000x reference)
import functools

import jax
import jax.numpy as jnp
from jax import lax
from jax.experimental import pallas as pl

D_MODEL = 768
N_HEADS = 12
DK = D_MODEL // N_HEADS
NEG_INF = -1e30


def _project(queries, keys, values, Wq, bq, Wk, bk, Wv, bv):
    B, L, D = queries.shape
    RB = 256
    nblk = L // RB
    grid = (B, nblk)
    wspec = pl.BlockSpec((D, D), lambda b, i: (0, 0))
    bspec = pl.BlockSpec((1, D), lambda b, i: (0, 0))
    xspec = pl.BlockSpec((1, RB, D), lambda b, i: (b, i, 0))
    ospec = pl.BlockSpec((1, N_HEADS, RB, DK), lambda b, i: (b, 0, i, 0))
    nspec = pl.BlockSpec((1, N_HEADS, RB), lambda b, i: (b, 0, i))

    def body(qin, kin, vin, wq, bq, wk, bk, wv, bv, qo, ko, vo, qn):
        xq = qin[0]
        xk = kin[0]
        xv = vin[0]
        q = jnp.dot(xq, wq[...], preferred_element_type=jnp.float32) + bq[...]
        k = jnp.dot(xk, wk[...], preferred_element_type=jnp.float32) + bk[...]
        v = jnp.dot(xv, wv[...], preferred_element_type=jnp.float32) + bv[...]
        qh = q.reshape(RB, N_HEADS, DK)
        kh = k.reshape(RB, N_HEADS, DK)
        vh = v.reshape(RB, N_HEADS, DK)
        qo[...] = jnp.transpose(qh, (1, 0, 2)).reshape(1, N_HEADS, RB, DK)
        ko[...] = jnp.transpose(kh, (1, 0, 2)).reshape(1, N_HEADS, RB, DK)
        vo[...] = jnp.transpose(vh, (1, 0, 2)).reshape(1, N_HEADS, RB, DK)
        nrm = jnp.sum(qh * qh, axis=-1)
        qn[...] = jnp.transpose(nrm, (1, 0)).reshape(1, N_HEADS, RB)

    out_shapes = (
        jax.ShapeDtypeStruct((B, N_HEADS, L, DK), jnp.float32),
        jax.ShapeDtypeStruct((B, N_HEADS, L, DK), jnp.float32),
        jax.ShapeDtypeStruct((B, N_HEADS, L, DK), jnp.float32),
        jax.ShapeDtypeStruct((B, N_HEADS, L), jnp.float32),
    )
    return pl.pallas_call(
        body,
        grid=grid,
        in_specs=[xspec, xspec, xspec,
                  wspec, bspec, wspec, bspec, wspec, bspec],
        out_specs=(ospec, ospec, ospec, nspec),
        out_shape=out_shapes,
    )(queries, keys, values, Wq, bq.reshape(1, D), Wk, bk.reshape(1, D),
      Wv, bv.reshape(1, D))


def _ranks(qn):
    B, H, L = qn.shape
    IB = 128
    nblk = L // IB
    grid = (B * H, nblk)

    def body(qn_ref, out_ref):
        bh = pl.program_id(0)
        ib = pl.program_id(1)
        b = bh // H
        h = bh % H
        n = qn_ref[b, h, :]
        ni = lax.dynamic_slice(n, (ib * IB,), (IB,))
        nj = n[None, :]
        nib = ni[:, None]
        gt = (nj > nib).astype(jnp.int32)
        jidx = lax.broadcasted_iota(jnp.int32, (IB, L), 1)
        iidx = lax.broadcasted_iota(jnp.int32, (IB, L), 0) + ib * IB
        tie = ((nj == nib) & (jidx < iidx)).astype(jnp.int32)
        out_ref[...] = jnp.sum(gt + tie, axis=1).reshape(1, 1, IB)

    return pl.pallas_call(
        body,
        grid=grid,
        in_specs=[pl.BlockSpec((B, H, L), lambda bh, ib: (0, 0, 0))],
        out_specs=pl.BlockSpec(
            (1, 1, IB), lambda bh, ib: (bh * (L // IB) + ib, 0, 0)),
        out_shape=jax.ShapeDtypeStruct((B * H * nblk, 1, IB), jnp.int32),
    )(qn)


def _masked_attention(q, k, v, ranks, U):
    B, H, L, dk = q.shape
    RB = 256
    nblk = L // RB
    grid = (B * H, nblk)
    scale = 1.0 / (dk ** 0.5)

    def body(q_ref, k_ref, v_ref, r_ref, o_ref):
        ib = pl.program_id(1)
        qb = q_ref[0, 0]
        kb = k_ref[0, 0]
        vb = v_ref[0, 0]
        r = r_ref[0, 0, :]
        scores = jnp.dot(qb, kb.T, preferred_element_type=jnp.float32) * scale
        colmask = (r < U)[None, :]
        scores = jnp.where(colmask, scores, NEG_INF)
        m = jnp.max(scores, axis=1, keepdims=True)
        p = jnp.exp(scores - m)
        s = jnp.sum(p, axis=1, keepdims=True)
        attn = p / s
        ctx = jnp.dot(attn, vb, preferred_element_type=jnp.float32)
        rsel = lax.dynamic_slice(r, (ib * RB,), (RB,)) < U
        ctx = jnp.where(rsel[:, None], ctx, 0.0)
        o_ref[...] = ctx.reshape(1, 1, RB, dk)

    qspec = pl.BlockSpec(
        (1, 1, RB, dk), lambda bh, ib: (bh // H, bh % H, ib, 0))
    kvspec = pl.BlockSpec(
        (1, 1, L, dk), lambda bh, ib: (bh // H, bh % H, 0, 0))
    rspec = pl.BlockSpec((1, 1, L), lambda bh, ib: (bh, 0, 0))
    return pl.pallas_call(
        body,
        grid=grid,
        in_specs=[qspec, kvspec, kvspec, rspec],
        out_specs=pl.BlockSpec(
            (1, 1, RB, dk), lambda bh, ib: (bh // H, bh % H, ib, 0)),
        out_shape=jax.ShapeDtypeStruct((B, H, L, dk), jnp.float32),
    )(q, k, v, ranks)


def _out_proj(ctx, Wo, bo):
    B, H, L, dk = ctx.shape
    D = H * dk
    RB = 256
    nblk = L // RB
    grid = (B, nblk)

    def body(c_ref, w_ref, b_ref, o_ref):
        c = c_ref[0]
        c2 = jnp.transpose(c, (1, 0, 2)).reshape(RB, D)
        o = jnp.dot(c2, w_ref[...], preferred_element_type=jnp.float32)
        o_ref[...] = (o + b_ref[...]).reshape(1, RB, D)

    return pl.pallas_call(
        body,
        grid=grid,
        in_specs=[
            pl.BlockSpec((1, H, RB, dk), lambda b, i: (b, 0, i, 0)),
            pl.BlockSpec((D, D), lambda b, i: (0, 0)),
            pl.BlockSpec((1, D), lambda b, i: (0, 0)),
        ],
        out_specs=pl.BlockSpec((1, RB, D), lambda b, i: (b, i, 0)),
        out_shape=jax.ShapeDtypeStruct((B, L, D), jnp.float32),
    )(ctx, Wo, bo.reshape(1, D))


@jax.jit
def kernel(queries, keys, values, Wq, bq, Wk, bk, Wv, bv, Wo, bo):
    B, L, D = queries.shape
    U = max(1, int(L * 0.5))
    q, k, v, qn = _project(queries, keys, values, Wq, bq, Wk, bk, Wv, bv)
    ranks = _ranks(qn)
    ranks = ranks.reshape(B * N_HEADS, 1, L)
    ctx = _masked_attention(q, k, v, ranks, U)
    return _out_proj(ctx, Wo, bo)


# speedup vs baseline: 1.4412x; 1.4412x over previous
import functools

import jax
import jax.numpy as jnp
from jax import lax
from jax.experimental import pallas as pl

D_MODEL = 768
N_HEADS = 12
DK = D_MODEL // N_HEADS
NEG_INF = -1e30


def _project(queries, keys, values, Wq, bq, Wk, bk, Wv, bv):
    B, L, D = queries.shape
    RB = 256
    nblk = L // RB
    grid = (B, nblk)
    wspec = pl.BlockSpec((D, D), lambda b, i: (0, 0))
    bspec = pl.BlockSpec((1, D), lambda b, i: (0, 0))
    xspec = pl.BlockSpec((1, RB, D), lambda b, i: (b, i, 0))
    ospec = pl.BlockSpec((1, N_HEADS, RB, DK), lambda b, i: (b, 0, i, 0))
    nspec = pl.BlockSpec((1, N_HEADS, RB), lambda b, i: (b, 0, i))

    def body(qin, kin, vin, wq, bq, wk, bk, wv, bv, qo, ko, vo, qn):
        xq = qin[0]
        xk = kin[0]
        xv = vin[0]
        q = jnp.dot(xq, wq[...], preferred_element_type=jnp.float32) + bq[...]
        k = jnp.dot(xk, wk[...], preferred_element_type=jnp.float32) + bk[...]
        v = jnp.dot(xv, wv[...], preferred_element_type=jnp.float32) + bv[...]
        qh = q.reshape(RB, N_HEADS, DK)
        kh = k.reshape(RB, N_HEADS, DK)
        vh = v.reshape(RB, N_HEADS, DK)
        qo[...] = jnp.transpose(qh, (1, 0, 2)).reshape(1, N_HEADS, RB, DK)
        ko[...] = jnp.transpose(kh, (1, 0, 2)).reshape(1, N_HEADS, RB, DK)
        vo[...] = jnp.transpose(vh, (1, 0, 2)).reshape(1, N_HEADS, RB, DK)
        nrm = jnp.sum(qh * qh, axis=-1)
        qn[...] = jnp.transpose(nrm, (1, 0)).reshape(1, N_HEADS, RB)

    out_shapes = (
        jax.ShapeDtypeStruct((B, N_HEADS, L, DK), jnp.float32),
        jax.ShapeDtypeStruct((B, N_HEADS, L, DK), jnp.float32),
        jax.ShapeDtypeStruct((B, N_HEADS, L, DK), jnp.float32),
        jax.ShapeDtypeStruct((B, N_HEADS, L), jnp.float32),
    )
    return pl.pallas_call(
        body,
        grid=grid,
        in_specs=[xspec, xspec, xspec,
                  wspec, bspec, wspec, bspec, wspec, bspec],
        out_specs=(ospec, ospec, ospec, nspec),
        out_shape=out_shapes,
    )(queries, keys, values, Wq, bq.reshape(1, D), Wk, bk.reshape(1, D),
      Wv, bv.reshape(1, D))


def _ranks(qn):
    BH, _, L = qn.shape
    IB = 128
    nblk = L // IB
    grid = (BH, nblk)

    def body(qn_ref, qni_ref, out_ref, col_ref):
        ib = pl.program_id(1)
        n = qn_ref[0, 0, :]
        ni = qni_ref[0, 0, :]
        nj = n[None, :]
        nib = ni[:, None]
        gt = (nj > nib).astype(jnp.int32)
        jidx = lax.broadcasted_iota(jnp.int32, (IB, L), 1)
        iidx = lax.broadcasted_iota(jnp.int32, (IB, L), 0) + ib * IB
        tie = ((nj == nib) & (jidx < iidx)).astype(jnp.int32)
        cnt = gt + tie
        out_ref[...] = jnp.sum(cnt, axis=1).reshape(1, 1, IB)
        col_ref[...] = jnp.sum(cnt, axis=1, keepdims=True).reshape(1, IB, 1)

    return pl.pallas_call(
        body,
        grid=grid,
        in_specs=[
            pl.BlockSpec((1, 1, L), lambda bh, ib: (bh, 0, 0)),
            pl.BlockSpec((1, 1, IB), lambda bh, ib: (bh, 0, ib)),
        ],
        out_specs=(
            pl.BlockSpec(
                (1, 1, IB), lambda bh, ib: (bh * (L // IB) + ib, 0, 0)),
            pl.BlockSpec(
                (1, IB, 1), lambda bh, ib: (bh * (L // IB) + ib, 0, 0)),
        ),
        out_shape=(
            jax.ShapeDtypeStruct((BH * nblk, 1, IB), jnp.int32),
            jax.ShapeDtypeStruct((BH * nblk, IB, 1), jnp.int32),
        ),
    )(qn, qn)


def _masked_attention(q, k, v, ranks, ranks_col, U):
    B, H, L, dk = q.shape
    RB = 256
    nblk = L // RB
    grid = (B * H, nblk)
    scale = 1.0 / (dk ** 0.5)

    def body(q_ref, k_ref, v_ref, r_ref, ri_ref, o_ref):
        qb = q_ref[0, 0]
        kb = k_ref[0, 0]
        vb = v_ref[0, 0]
        r = r_ref[0, 0, :]
        scores = jnp.dot(qb, kb.T, preferred_element_type=jnp.float32) * scale
        colmask = (r < U)[None, :]
        scores = jnp.where(colmask, scores, NEG_INF)
        m = jnp.max(scores, axis=1, keepdims=True)
        p = jnp.exp(scores - m)
        s = jnp.sum(p, axis=1, keepdims=True)
        attn = p / s
        ctx = jnp.dot(attn, vb, preferred_element_type=jnp.float32)
        rsel = ri_ref[0] < U
        ctx = jnp.where(rsel, ctx, 0.0)
        o_ref[...] = ctx.reshape(1, 1, RB, dk)

    qspec = pl.BlockSpec(
        (1, 1, RB, dk), lambda bh, ib: (bh // H, bh % H, ib, 0))
    kvspec = pl.BlockSpec(
        (1, 1, L, dk), lambda bh, ib: (bh // H, bh % H, 0, 0))
    rspec = pl.BlockSpec((1, 1, L), lambda bh, ib: (bh, 0, 0))
    rispec = pl.BlockSpec((1, RB, 1), lambda bh, ib: (bh, ib, 0))
    return pl.pallas_call(
        body,
        grid=grid,
        in_specs=[qspec, kvspec, kvspec, rspec, rispec],
        out_specs=pl.BlockSpec(
            (1, 1, RB, dk), lambda bh, ib: (bh // H, bh % H, ib, 0)),
        out_shape=jax.ShapeDtypeStruct((B, H, L, dk), jnp.float32),
    )(q, k, v, ranks, ranks_col)


def _out_proj(ctx, Wo, bo):
    B, H, L, dk = ctx.shape
    D = H * dk
    RB = 256
    nblk = L // RB
    grid = (B, nblk)

    def body(c_ref, w_ref, b_ref, o_ref):
        c = c_ref[0]
        c2 = jnp.transpose(c, (1, 0, 2)).reshape(RB, D)
        o = jnp.dot(c2, w_ref[...], preferred_element_type=jnp.float32)
        o_ref[...] = (o + b_ref[...]).reshape(1, RB, D)

    return pl.pallas_call(
        body,
        grid=grid,
        in_specs=[
            pl.BlockSpec((1, H, RB, dk), lambda b, i: (b, 0, i, 0)),
            pl.BlockSpec((D, D), lambda b, i: (0, 0)),
            pl.BlockSpec((1, D), lambda b, i: (0, 0)),
        ],
        out_specs=pl.BlockSpec((1, RB, D), lambda b, i: (b, i, 0)),
        out_shape=jax.ShapeDtypeStruct((B, L, D), jnp.float32),
    )(ctx, Wo, bo.reshape(1, D))


@jax.jit
def kernel(queries, keys, values, Wq, bq, Wk, bk, Wv, bv, Wo, bo):
    B, L, D = queries.shape
    U = max(1, int(L * 0.5))
    q, k, v, qn = _project(queries, keys, values, Wq, bq, Wk, bk, Wv, bv)
    ranks, ranks_col = _ranks(qn.reshape(B * N_HEADS, 1, L))
    ranks = ranks.reshape(B * N_HEADS, 1, L)
    ranks_col = ranks_col.reshape(B * N_HEADS, L, 1)
    ctx = _masked_attention(q, k, v, ranks, ranks_col, U)
    return _out_proj(ctx, Wo, bo)


# speedup vs baseline: 3.7051x; 2.5708x over previous
import functools

import jax
import jax.numpy as jnp
from jax import lax
from jax.experimental import pallas as pl
from jax.experimental.pallas import tpu as pltpu
from jax.experimental.pallas import tpu_sc as plsc

D_MODEL = 768
N_HEADS = 12
DK = D_MODEL // N_HEADS


def _project(queries, keys, values, Wq, bq, Wk, bk, Wv, bv):
    B, L, D = queries.shape
    RB = 256
    nblk = L // RB
    grid = (B, nblk)
    wspec = pl.BlockSpec((D, D), lambda b, i: (0, 0))
    bspec = pl.BlockSpec((1, D), lambda b, i: (0, 0))
    xspec = pl.BlockSpec((1, RB, D), lambda b, i: (b, i, 0))
    ospec = pl.BlockSpec((1, N_HEADS, RB, 256), lambda b, i: (b, 0, i, 0))
    nspec = pl.BlockSpec((1, N_HEADS, RB), lambda b, i: (b, 0, i))

    def body(qin, kin, vin, wq, bq, wk, bk, wv, bv, qo, qn):
        xq = qin[0]
        xk = kin[0]
        xv = vin[0]
        q = jnp.dot(xq, wq[...], preferred_element_type=jnp.float32) + bq[...]
        k = jnp.dot(xk, wk[...], preferred_element_type=jnp.float32) + bk[...]
        v = jnp.dot(xv, wv[...], preferred_element_type=jnp.float32) + bv[...]
        qh = q.reshape(RB, N_HEADS, DK)
        kh = k.reshape(RB, N_HEADS, DK)
        vh = v.reshape(RB, N_HEADS, DK)
        qkv = jnp.concatenate([qh, kh, vh], axis=-1)
        qo[0, :, :, :3 * DK] = jnp.transpose(qkv, (1, 0, 2))
        nrm = jnp.sum(qh * qh, axis=-1)
        qn[...] = jnp.transpose(nrm, (1, 0)).reshape(1, N_HEADS, RB)

    out_shapes = (
        jax.ShapeDtypeStruct((B, N_HEADS, L, 256), jnp.float32),
        jax.ShapeDtypeStruct((B, N_HEADS, L), jnp.float32),
    )
    return pl.pallas_call(
        body,
        grid=grid,
        in_specs=[xspec, xspec, xspec,
                  wspec, bspec, wspec, bspec, wspec, bspec],
        out_specs=(ospec, nspec),
        out_shape=out_shapes,
    )(queries, keys, values, Wq, bq.reshape(1, D), Wk, bk.reshape(1, D),
      Wv, bv.reshape(1, D))


def _select(qn, U):
    BH, L = qn.shape

    def body(qn_ref, out_ref):
        keys = lax.bitcast_convert_type(qn_ref[...], jnp.int32)

        def bisect(t, prefix):
            cand = prefix | jnp.left_shift(1, 30 - t)
            cnt = jnp.sum((keys >= cand).astype(jnp.int32), axis=1,
                          keepdims=True)
            return jnp.where(cnt >= U, cand, prefix)

        thr = lax.fori_loop(0, 31, bisect, jnp.zeros((BH, 1), jnp.int32))
        gtT = keys > thr
        eqT = keys == thr
        c_gt = jnp.sum(gtT.astype(jnp.int32), axis=1, keepdims=True)
        need = U - c_gt

        jidx = lax.broadcasted_iota(jnp.int32, (L, L), 0)
        lidx = lax.broadcasted_iota(jnp.int32, (L, L), 1)
        lt = jnp.where(jidx < lidx, 1.0, 0.0)

        tp = jnp.dot(eqT.astype(jnp.float32), lt,
                     preferred_element_type=jnp.float32)
        sel = gtT | (eqT & (tp.astype(jnp.int32) < need))
        sp = jnp.dot(sel.astype(jnp.float32), lt,
                     preferred_element_type=jnp.float32).astype(jnp.int32)
        lrow = lax.broadcasted_iota(jnp.int32, (BH, L), 1)
        out_ref[...] = jnp.where(sel, sp, U + lrow - sp)

    return pl.pallas_call(
        body,
        in_specs=[pl.BlockSpec((BH, L), lambda: (0, 0))],
        out_specs=pl.BlockSpec((BH, L), lambda: (0, 0)),
        out_shape=jax.ShapeDtypeStruct((BH, L), jnp.int32),
    )(qn)


def _inv_sel(ranks, U):
    BH, _, L = ranks.shape
    IB = 128
    nblk = U // IB
    grid = (BH, nblk)

    def body(r_ref, out_ref):
        ub = pl.program_id(1)
        r = r_ref[0, 0, :]
        u_iota = lax.broadcasted_iota(jnp.int32, (IB, L), 0) + ub * IB
        eq = (r[None, :] == u_iota).astype(jnp.float32)
        jf = lax.broadcasted_iota(jnp.int32, (IB, L), 1).astype(jnp.float32)
        inv = jnp.sum(eq * jf, axis=1).astype(jnp.int32)
        out_ref[...] = inv.reshape(1, 1, IB)

    return pl.pallas_call(
        body,
        grid=grid,
        in_specs=[pl.BlockSpec((1, 1, L), lambda bh, ub: (bh, 0, 0))],
        out_specs=pl.BlockSpec(
            (1, 1, IB), lambda bh, ub: (bh * (U // IB) + ub, 0, 0)),
        out_shape=jax.ShapeDtypeStruct((BH * nblk, 1, IB), jnp.int32),
    )(ranks)


def _sc_gather_qkv(lsel, qkv, B, H, L, U):
    BH = B * H
    HU = U // 2
    ST = 256
    mesh = plsc.VectorSubcoreMesh(core_axis_name="c", subcore_axis_name="s")
    out_type = jax.ShapeDtypeStruct((BH, U, 256), jnp.float32)

    @functools.partial(
        pl.kernel,
        mesh=mesh,
        out_type=out_type,
        scratch_types=[
            pltpu.VMEM((HU,), jnp.int32),
            pltpu.VMEM((ST, 256), jnp.float32),
            pltpu.SemaphoreType.DMA,
        ],
    )
    def k(lsel_hbm, qkv_hbm, qs_hbm, lsel_v, rows_v, sem):
        cid = lax.axis_index("c")
        sid = lax.axis_index("s")
        wid = sid * 2 + cid
        for t in (wid, wid + 32):
            @pl.when(t < 2 * BH)
            def _():
                bh = t // 2
                half = t % 2
                pltpu.sync_copy(lsel_hbm.at[bh, pl.ds(half * HU, HU)],
                                lsel_v)
                for rnd in range(HU // ST):
                    cps = [
                        pltpu.async_copy(
                            qkv_hbm.at[bh].at[
                                lsel_v.at[pl.ds(rnd * ST + c * 128, 128)]],
                            rows_v.at[pl.ds(c * 128, 128)], sem)
                        for c in range(ST // 128)
                    ]
                    for cp in cps:
                        cp.wait()
                    pltpu.sync_copy(
                        rows_v,
                        qs_hbm.at[bh, pl.ds(half * HU + rnd * ST, ST)])

    return k(lsel, qkv)


def _sparse_attention(qs, L):
    BH, U, _ = qs.shape
    dk = DK
    RB = 512
    nbu = U // RB
    nblk = L // RB
    grid = (BH, nblk)
    scale = 1.0 / (dk ** 0.5)

    def body(kv_ref, o_ref):
        ib = pl.program_id(1)

        @pl.when(ib < nbu)
        def _():
            qb = kv_ref[0, pl.ds(ib * RB, RB), :dk]
            kb = kv_ref[0, :, dk:2 * dk]
            vb = kv_ref[0, :, 2 * dk:3 * dk]
            scores = lax.dot_general(
                qb, kb, (((1,), (1,)), ((), ())),
                preferred_element_type=jnp.float32) * scale
            m = jnp.max(scores, axis=1, keepdims=True)
            p = jnp.exp(scores - m)
            ssum = jnp.sum(p, axis=1, keepdims=True)
            attn = p / ssum
            ctx = jnp.dot(attn, vb, preferred_element_type=jnp.float32)
            o_ref[0, :, :dk] = ctx

        @pl.when(ib >= nbu)
        def _():
            o_ref[0, :, :dk] = jnp.zeros((RB, dk), jnp.float32)

    return pl.pallas_call(
        body,
        grid=grid,
        in_specs=[
            pl.BlockSpec((1, U, 256), lambda bh, ib: (bh, 0, 0)),
        ],
        out_specs=pl.BlockSpec((1, RB, 128), lambda bh, ib: (bh, ib, 0)),
        out_shape=jax.ShapeDtypeStruct((BH, L, 128), jnp.float32),
    )(qs)


def _sc_unscatter(slots, ctx_aug, B, H, L, U):
    BH = B * H
    HL = L // 2
    ST = 256
    mesh = plsc.VectorSubcoreMesh(core_axis_name="c", subcore_axis_name="s")
    out_type = jax.ShapeDtypeStruct((BH, L, 128), jnp.float32)

    @functools.partial(
        pl.kernel,
        mesh=mesh,
        out_type=out_type,
        scratch_types=[
            pltpu.VMEM((HL,), jnp.int32),
            pltpu.VMEM((ST, 128), jnp.float32),
            pltpu.SemaphoreType.DMA,
        ],
    )
    def k(slots_hbm, ctx_hbm, out_hbm, slots_v, rows_v, sem):
        cid = lax.axis_index("c")
        sid = lax.axis_index("s")
        wid = sid * 2 + cid
        for t in (wid, wid + 32):
            @pl.when(t < 2 * BH)
            def _():
                bh = t // 2
                half = t % 2
                pltpu.sync_copy(slots_hbm.at[bh, pl.ds(half * HL, HL)],
                                slots_v)
                for rnd in range(HL // ST):
                    cps = [
                        pltpu.async_copy(
                            ctx_hbm.at[bh].at[
                                slots_v.at[pl.ds(rnd * ST + c * 128, 128)]],
                            rows_v.at[pl.ds(c * 128, 128)], sem)
                        for c in range(ST // 128)
                    ]
                    for cp in cps:
                        cp.wait()
                    pltpu.sync_copy(
                        rows_v,
                        out_hbm.at[bh, pl.ds(half * HL + rnd * ST, ST)])

    return k(slots, ctx_aug)


def _out_proj(ctx, Wo, bo):
    B, H, L, _ = ctx.shape
    dk = DK
    D = H * dk
    RB = 256
    nblk = L // RB
    grid = (B, nblk)

    def body(c_ref, w_ref, b_ref, o_ref):
        c = c_ref[0, :, :, :dk]
        c2 = jnp.transpose(c, (1, 0, 2)).reshape(RB, D)
        o = jnp.dot(c2, w_ref[...], preferred_element_type=jnp.float32)
        o_ref[...] = (o + b_ref[...]).reshape(1, RB, D)

    return pl.pallas_call(
        body,
        grid=grid,
        in_specs=[
            pl.BlockSpec((1, H, RB, 128), lambda b, i: (b, 0, i, 0)),
            pl.BlockSpec((D, D), lambda b, i: (0, 0)),
            pl.BlockSpec((1, D), lambda b, i: (0, 0)),
        ],
        out_specs=pl.BlockSpec((1, RB, D), lambda b, i: (b, i, 0)),
        out_shape=jax.ShapeDtypeStruct((B, L, D), jnp.float32),
    )(ctx, Wo, bo.reshape(1, D))


@jax.jit
def kernel(queries, keys, values, Wq, bq, Wk, bk, Wv, bv, Wo, bo):
    B, L, D = queries.shape
    H = N_HEADS
    U = max(1, int(L * 0.5))
    qkv, qn = _project(queries, keys, values, Wq, bq, Wk, bk, Wv, bv)
    slots = _select(qn.reshape(B * H, L), U)
    lsel = _inv_sel(slots.reshape(B * H, 1, L), U).reshape(B * H, U)
    qs = _sc_gather_qkv(lsel, qkv.reshape(B * H, L, 256), B, H, L, U)
    ctx_aug = _sparse_attention(qs, L)
    ctx_full = _sc_unscatter(slots, ctx_aug, B, H, L, U)
    return _out_proj(ctx_full.reshape(B, H, L, 128), Wo, bo)


# speedup vs baseline: 3.9285x; 1.0603x over previous
import functools

import jax
import jax.numpy as jnp
from jax import lax
from jax.experimental import pallas as pl
from jax.experimental.pallas import tpu as pltpu
from jax.experimental.pallas import tpu_sc as plsc

D_MODEL = 768
N_HEADS = 12
DK = D_MODEL // N_HEADS


def _project(queries, keys, values, Wq, bq, Wk, bk, Wv, bv):
    B, L, D = queries.shape
    RB = 256
    nblk = L // RB
    grid = (B, nblk)
    wspec = pl.BlockSpec((D, D), lambda b, i: (0, 0))
    bspec = pl.BlockSpec((1, D), lambda b, i: (0, 0))
    xspec = pl.BlockSpec((1, RB, D), lambda b, i: (b, i, 0))
    ospec = pl.BlockSpec((1, N_HEADS, RB, 256), lambda b, i: (b, 0, i, 0))
    nspec = pl.BlockSpec((1, N_HEADS, RB), lambda b, i: (b, 0, i))

    def body(qin, kin, vin, wq, bq, wk, bk, wv, bv, qo, qn):
        xq = qin[0]
        xk = kin[0]
        xv = vin[0]
        q = jnp.dot(xq, wq[...], preferred_element_type=jnp.float32) + bq[...]
        k = jnp.dot(xk, wk[...], preferred_element_type=jnp.float32) + bk[...]
        v = jnp.dot(xv, wv[...], preferred_element_type=jnp.float32) + bv[...]
        qh = q.reshape(RB, N_HEADS, DK)
        kh = k.reshape(RB, N_HEADS, DK)
        vh = v.reshape(RB, N_HEADS, DK)
        qkv = jnp.concatenate([qh, kh, vh], axis=-1)
        qo[0, :, :, :3 * DK] = jnp.transpose(qkv, (1, 0, 2))
        nrm = jnp.sum(qh * qh, axis=-1)
        qn[...] = jnp.transpose(nrm, (1, 0)).reshape(1, N_HEADS, RB)

    out_shapes = (
        jax.ShapeDtypeStruct((B, N_HEADS, L, 256), jnp.float32),
        jax.ShapeDtypeStruct((B, N_HEADS, L), jnp.float32),
    )
    return pl.pallas_call(
        body,
        grid=grid,
        in_specs=[xspec, xspec, xspec,
                  wspec, bspec, wspec, bspec, wspec, bspec],
        out_specs=(ospec, nspec),
        out_shape=out_shapes,
    )(queries, keys, values, Wq, bq.reshape(1, D), Wk, bk.reshape(1, D),
      Wv, bv.reshape(1, D))


def _select(qn, U):
    BH, L = qn.shape

    def body(qn_ref, out_ref):
        keys = lax.bitcast_convert_type(qn_ref[...], jnp.int32)

        def bisect(t, prefix):
            cand = prefix | jnp.left_shift(1, 30 - t)
            cnt = jnp.sum((keys >= cand).astype(jnp.int32), axis=1,
                          keepdims=True)
            return jnp.where(cnt >= U, cand, prefix)

        thr = lax.fori_loop(0, 31, bisect, jnp.zeros((BH, 1), jnp.int32))
        gtT = keys > thr
        eqT = keys == thr
        c_gt = jnp.sum(gtT.astype(jnp.int32), axis=1, keepdims=True)
        need = U - c_gt

        jidx = lax.broadcasted_iota(jnp.int32, (L, L), 0)
        lidx = lax.broadcasted_iota(jnp.int32, (L, L), 1)
        lt = jnp.where(jidx < lidx, 1.0, 0.0)

        tp = jnp.dot(eqT.astype(jnp.float32), lt,
                     preferred_element_type=jnp.float32)
        sel = gtT | (eqT & (tp.astype(jnp.int32) < need))
        sp = jnp.dot(sel.astype(jnp.float32), lt,
                     preferred_element_type=jnp.float32).astype(jnp.int32)
        lrow = lax.broadcasted_iota(jnp.int32, (BH, L), 1)
        out_ref[...] = jnp.where(sel, sp, U + lrow - sp)

    return pl.pallas_call(
        body,
        in_specs=[pl.BlockSpec((BH, L), lambda: (0, 0))],
        out_specs=pl.BlockSpec((BH, L), lambda: (0, 0)),
        out_shape=jax.ShapeDtypeStruct((BH, L), jnp.int32),
    )(qn)


def _inv_sel(slots, U):
    BH, _, L = slots.shape
    grid = (BH,)

    def body(r_ref, out_ref):
        r = r_ref[0, 0, :]
        u_iota = lax.broadcasted_iota(jnp.int32, (U, L), 0)
        eq = jnp.where(r[None, :] == u_iota, 1.0, 0.0)
        jf = lax.broadcasted_iota(jnp.int32, (U, L), 1).astype(jnp.float32)
        inv = jnp.sum(eq * jf, axis=1, keepdims=True)
        out_ref[...] = inv.astype(jnp.int32).reshape(1, U, 1)

    return pl.pallas_call(
        body,
        grid=grid,
        in_specs=[pl.BlockSpec((1, 1, L), lambda bh: (bh, 0, 0))],
        out_specs=pl.BlockSpec((1, U, 1), lambda bh: (bh, 0, 0)),
        out_shape=jax.ShapeDtypeStruct((BH, U, 1), jnp.int32),
    )(slots)


def _sc_gather_qkv(lsel, qkv, B, H, L, U):
    BH = B * H
    HU = U // 2
    ST = 256
    mesh = plsc.VectorSubcoreMesh(core_axis_name="c", subcore_axis_name="s")
    out_type = jax.ShapeDtypeStruct((BH, U, 256), jnp.float32)

    @functools.partial(
        pl.kernel,
        mesh=mesh,
        out_type=out_type,
        scratch_types=[
            pltpu.VMEM((HU,), jnp.int32),
            pltpu.VMEM((ST, 256), jnp.float32),
            pltpu.SemaphoreType.DMA,
        ],
    )
    def k(lsel_hbm, qkv_hbm, qs_hbm, lsel_v, rows_v, sem):
        cid = lax.axis_index("c")
        sid = lax.axis_index("s")
        wid = sid * 2 + cid
        for t in (wid, wid + 32):
            @pl.when(t < 2 * BH)
            def _():
                bh = t // 2
                half = t % 2
                pltpu.sync_copy(lsel_hbm.at[bh, pl.ds(half * HU, HU)],
                                lsel_v)
                for rnd in range(HU // ST):
                    cps = [
                        pltpu.async_copy(
                            qkv_hbm.at[bh].at[
                                lsel_v.at[pl.ds(rnd * ST + c * 128, 128)]],
                            rows_v.at[pl.ds(c * 128, 128)], sem)
                        for c in range(ST // 128)
                    ]
                    for cp in cps:
                        cp.wait()
                    pltpu.sync_copy(
                        rows_v,
                        qs_hbm.at[bh, pl.ds(half * HU + rnd * ST, ST)])

    return k(lsel, qkv)


def _sparse_attention(qs, L):
    BH, U, _ = qs.shape
    dk = DK
    RB = 1024
    nbu = U // RB
    nblk = L // RB
    grid = (BH, nblk)
    scale = 1.0 / (dk ** 0.5)

    def body(kv_ref, o_ref):
        ib = pl.program_id(1)

        @pl.when(ib < nbu)
        def _():
            qb = kv_ref[0, pl.ds(ib * RB, RB), :dk]
            kb = kv_ref[0, :, dk:2 * dk]
            vb = kv_ref[0, :, 2 * dk:3 * dk]
            scores = lax.dot_general(
                qb, kb, (((1,), (1,)), ((), ())),
                preferred_element_type=jnp.float32) * scale
            m = jnp.max(scores, axis=1, keepdims=True)
            p = jnp.exp(scores - m)
            ssum = jnp.sum(p, axis=1, keepdims=True)
            attn = p / ssum
            ctx = jnp.dot(attn, vb, preferred_element_type=jnp.float32)
            o_ref[0, :, :dk] = ctx

        @pl.when(ib >= nbu)
        def _():
            o_ref[0, :, :dk] = jnp.zeros((RB, dk), jnp.float32)

    return pl.pallas_call(
        body,
        grid=grid,
        in_specs=[
            pl.BlockSpec((1, U, 256), lambda bh, ib: (bh, 0, 0)),
        ],
        out_specs=pl.BlockSpec((1, RB, 128), lambda bh, ib: (bh, ib, 0)),
        out_shape=jax.ShapeDtypeStruct((BH, L, 128), jnp.float32),
    )(qs)


def _sc_unscatter(slots, ctx_aug, B, H, L, U):
    BH = B * H
    HL = L // 2
    ST = 256
    mesh = plsc.VectorSubcoreMesh(core_axis_name="c", subcore_axis_name="s")
    out_type = jax.ShapeDtypeStruct((BH, L, 128), jnp.float32)

    @functools.partial(
        pl.kernel,
        mesh=mesh,
        out_type=out_type,
        scratch_types=[
            pltpu.VMEM((HL,), jnp.int32),
            pltpu.VMEM((ST, 128), jnp.float32),
            pltpu.SemaphoreType.DMA,
        ],
    )
    def k(slots_hbm, ctx_hbm, out_hbm, slots_v, rows_v, sem):
        cid = lax.axis_index("c")
        sid = lax.axis_index("s")
        wid = sid * 2 + cid
        for t in (wid, wid + 32):
            @pl.when(t < 2 * BH)
            def _():
                bh = t // 2
                half = t % 2
                pltpu.sync_copy(slots_hbm.at[bh, pl.ds(half * HL, HL)],
                                slots_v)
                for rnd in range(HL // ST):
                    cps = [
                        pltpu.async_copy(
                            ctx_hbm.at[bh].at[
                                slots_v.at[pl.ds(rnd * ST + c * 128, 128)]],
                            rows_v.at[pl.ds(c * 128, 128)], sem)
                        for c in range(ST // 128)
                    ]
                    for cp in cps:
                        cp.wait()
                    pltpu.sync_copy(
                        rows_v,
                        out_hbm.at[bh, pl.ds(half * HL + rnd * ST, ST)])

    return k(slots, ctx_aug)


def _out_proj(ctx, Wo, bo):
    B, H, L, _ = ctx.shape
    dk = DK
    D = H * dk
    RB = 256
    nblk = L // RB
    grid = (B, nblk)

    def body(c_ref, w_ref, b_ref, o_ref):
        c = c_ref[0, :, :, :dk]
        c2 = jnp.transpose(c, (1, 0, 2)).reshape(RB, D)
        o = jnp.dot(c2, w_ref[...], preferred_element_type=jnp.float32)
        o_ref[...] = (o + b_ref[...]).reshape(1, RB, D)

    return pl.pallas_call(
        body,
        grid=grid,
        in_specs=[
            pl.BlockSpec((1, H, RB, 128), lambda b, i: (b, 0, i, 0)),
            pl.BlockSpec((D, D), lambda b, i: (0, 0)),
            pl.BlockSpec((1, D), lambda b, i: (0, 0)),
        ],
        out_specs=pl.BlockSpec((1, RB, D), lambda b, i: (b, i, 0)),
        out_shape=jax.ShapeDtypeStruct((B, L, D), jnp.float32),
    )(ctx, Wo, bo.reshape(1, D))


@jax.jit
def kernel(queries, keys, values, Wq, bq, Wk, bk, Wv, bv, Wo, bo):
    B, L, D = queries.shape
    H = N_HEADS
    U = max(1, int(L * 0.5))
    qkv, qn = _project(queries, keys, values, Wq, bq, Wk, bk, Wv, bv)
    slots = _select(qn.reshape(B * H, L), U)
    lsel = _inv_sel(slots.reshape(B * H, 1, L), U).reshape(B * H, U)
    qs = _sc_gather_qkv(lsel, qkv.reshape(B * H, L, 256), B, H, L, U)
    ctx_aug = _sparse_attention(qs, L)
    ctx_full = _sc_unscatter(slots, ctx_aug, B, H, L, U)
    return _out_proj(ctx_full.reshape(B, H, L, 128), Wo, bo)
